# baseline (speedup 1.0000x reference)
import functools

import jax
import jax.numpy as jnp
from jax import lax
from jax.experimental import pallas as pl
from jax.experimental.pallas import tpu as pltpu
from jax.experimental.pallas import tpu_sc as plsc

B = 16384
EMBED = 32
NEG = 8
NC = 2
NS = 16
NW = NC * NS
BPW = B // NW
C = 128
NCHUNK = BPW // C
H = EMBED // 2


def _mf_body(user_hbm, item_hbm, users_hbm, items_hbm, negidx_hbm,
             pos_hbm, neg_hbm,
             uidx, iidx, nidx, u_rows, it_rows, ineg_rows,
             pos_buf, neg_buf, sem):
    wid = lax.axis_index("s") * NC + lax.axis_index("c")
    base = wid * BPW
    lane = lax.broadcasted_iota(jnp.int32, (H,), 0)
    last = lane == (H - 1)

    for c in range(NCHUNK):
        cbase = base + c * C
        pltpu.sync_copy(users_hbm.at[pl.ds(cbase, C)], uidx)
        pltpu.sync_copy(items_hbm.at[pl.ds(cbase, C)], iidx)
        pltpu.sync_copy(negidx_hbm.at[pl.ds(cbase * NEG, C * NEG)], nidx)
        copies = [
            pltpu.async_copy(user_hbm.at[uidx], u_rows, sem),
            pltpu.async_copy(item_hbm.at[iidx], it_rows, sem),
        ]
        for k in range(NEG):
            copies.append(pltpu.async_copy(
                item_hbm.at[nidx.at[pl.ds(k * C, C)]],
                ineg_rows.at[pl.ds(k * C, C)], sem))
        for cp in copies:
            cp.wait()

        def elem_body(e, _):
            u0 = u_rows[e, pl.ds(0, H)]
            u1 = u_rows[e, pl.ds(H, H)]
            i0 = it_rows[e, pl.ds(0, H)]
            i1 = it_rows[e, pl.ds(H, H)]
            ps = plsc.cumsum(u0 * i0 + u1 * i1)
            plsc.store_scatter(
                pos_buf, [jnp.full((H,), c * C + e, jnp.int32)], ps,
                mask=last)

            def neg_body(j, _):
                r = e * NEG + j
                n0 = ineg_rows[r, pl.ds(0, H)]
                n1 = ineg_rows[r, pl.ds(H, H)]
                ns = plsc.cumsum(u0 * n0 + u1 * n1)
                plsc.store_scatter(
                    neg_buf,
                    [jnp.full((H,), (c * C + e) * NEG + j, jnp.int32)], ns,
                    mask=last)
                return 0

            lax.fori_loop(0, NEG, neg_body, 0)
            return 0

        lax.fori_loop(0, C, elem_body, 0)

    pltpu.sync_copy(pos_buf, pos_hbm.at[pl.ds(base, BPW)])
    pltpu.sync_copy(neg_buf, neg_hbm.at[pl.ds(base * NEG, BPW * NEG)])


_mf = pl.kernel(
    _mf_body,
    out_type=(
        jax.ShapeDtypeStruct((B,), jnp.float32),
        jax.ShapeDtypeStruct((B * NEG,), jnp.float32),
    ),
    mesh=plsc.VectorSubcoreMesh(
        core_axis_name="c", subcore_axis_name="s",
        num_cores=NC, num_subcores=NS),
    scratch_types=[
        pltpu.VMEM((C,), jnp.int32),
        pltpu.VMEM((C,), jnp.int32),
        pltpu.VMEM((C * NEG,), jnp.int32),
        pltpu.VMEM((C, EMBED), jnp.float32),
        pltpu.VMEM((C, EMBED), jnp.float32),
        pltpu.VMEM((C * NEG, EMBED), jnp.float32),
        pltpu.VMEM((BPW,), jnp.float32),
        pltpu.VMEM((BPW * NEG,), jnp.float32),
        pltpu.SemaphoreType.DMA,
    ],
    compiler_params=pltpu.CompilerParams(
        needs_layout_passes=False, use_tc_tiling_on_sc=False),
)


def kernel(user_embeds, item_embeds, users, items, items_neg):
    users = users.astype(jnp.int32)
    items = items.astype(jnp.int32)
    neg_flat = items_neg.astype(jnp.int32).reshape(B * NEG)
    pos, neg = _mf(user_embeds, item_embeds, users, items, neg_flat)
    return pos, neg.reshape(B, NEG)


# speedup vs baseline: 1.0045x; 1.0045x over previous
import functools

import jax
import jax.numpy as jnp
from jax import lax
from jax.experimental import pallas as pl
from jax.experimental.pallas import tpu as pltpu
from jax.experimental.pallas import tpu_sc as plsc

B = 16384
EMBED = 32
NEG = 8
NC = 2
NS = 16
NW = NC * NS
BPW = B // NW
C = 128
NCHUNK = BPW // C
H = EMBED // 2


def _mf_body(user_hbm, item_hbm, users_hbm, items_hbm, negidx_hbm,
             pos_hbm, neg_hbm,
             uidx, iidx, nidx, u_rows, it_rows, ineg_rows,
             pos_buf, neg_buf, sem):
    wid = lax.axis_index("s") * NC + lax.axis_index("c")
    base = wid * BPW
    lane = lax.broadcasted_iota(jnp.int32, (H,), 0)
    last = lane == (H - 1)

    for c in range(NCHUNK):
        cbase = base + c * C
        pltpu.sync_copy(users_hbm.at[pl.ds(cbase, C)], uidx)
        pltpu.sync_copy(items_hbm.at[pl.ds(cbase, C)], iidx)
        pltpu.sync_copy(negidx_hbm.at[pl.ds(cbase * NEG, C * NEG)], nidx)
        copies = [
            pltpu.async_copy(user_hbm.at[uidx], u_rows, sem),
            pltpu.async_copy(item_hbm.at[iidx], it_rows, sem),
        ]
        for k in range(NEG):
            copies.append(pltpu.async_copy(
                item_hbm.at[nidx.at[pl.ds(k * C, C)]],
                ineg_rows.at[pl.ds(k * C, C)], sem))
        for cp in copies:
            cp.wait()

        def elem_body(e, _):
            u0 = u_rows[e, pl.ds(0, H)]
            u1 = u_rows[e, pl.ds(H, H)]
            i0 = it_rows[e, pl.ds(0, H)]
            i1 = it_rows[e, pl.ds(H, H)]
            ps = plsc.cumsum(u0 * i0 + u1 * i1)
            plsc.store_scatter(
                pos_buf, [jnp.full((H,), c * C + e, jnp.int32)], ps,
                mask=last)

            for j in range(NEG):
                r = e * NEG + j
                n0 = ineg_rows[r, pl.ds(0, H)]
                n1 = ineg_rows[r, pl.ds(H, H)]
                ns = plsc.cumsum(u0 * n0 + u1 * n1)
                plsc.store_scatter(
                    neg_buf,
                    [jnp.full((H,), (c * C + e) * NEG + j, jnp.int32)], ns,
                    mask=last)
            return 0

        lax.fori_loop(0, C, elem_body, 0, unroll=2)

    pltpu.sync_copy(pos_buf, pos_hbm.at[pl.ds(base, BPW)])
    pltpu.sync_copy(neg_buf, neg_hbm.at[pl.ds(base * NEG, BPW * NEG)])


_mf = pl.kernel(
    _mf_body,
    out_type=(
        jax.ShapeDtypeStruct((B,), jnp.float32),
        jax.ShapeDtypeStruct((B * NEG,), jnp.float32),
    ),
    mesh=plsc.VectorSubcoreMesh(
        core_axis_name="c", subcore_axis_name="s",
        num_cores=NC, num_subcores=NS),
    scratch_types=[
        pltpu.VMEM((C,), jnp.int32),
        pltpu.VMEM((C,), jnp.int32),
        pltpu.VMEM((C * NEG,), jnp.int32),
        pltpu.VMEM((C, EMBED), jnp.float32),
        pltpu.VMEM((C, EMBED), jnp.float32),
        pltpu.VMEM((C * NEG, EMBED), jnp.float32),
        pltpu.VMEM((BPW,), jnp.float32),
        pltpu.VMEM((BPW * NEG,), jnp.float32),
        pltpu.SemaphoreType.DMA,
    ],
    compiler_params=pltpu.CompilerParams(
        needs_layout_passes=False, use_tc_tiling_on_sc=False),
)


def kernel(user_embeds, item_embeds, users, items, items_neg):
    users = users.astype(jnp.int32)
    items = items.astype(jnp.int32)
    neg_flat = items_neg.astype(jnp.int32).reshape(B * NEG)
    pos, neg = _mf(user_embeds, item_embeds, users, items, neg_flat)
    return pos, neg.reshape(B, NEG)
